# baseline (speedup 1.0000x reference)
import functools

import jax
import jax.numpy as jnp
from jax import lax
from jax.experimental import pallas as pl
from jax.experimental.pallas import tpu as pltpu
from jax.experimental.pallas import tpu_sc as plsc

B = 16384
D = 64

NC = 2
NS = 16
NW = NC * NS
L = 16

BPW = B // NW
GATHER_CHUNK = 128
N_CHUNKS = BPW // GATHER_CHUNK
N_GROUPS = BPW // L


@functools.partial(
    pl.kernel,
    mesh=plsc.VectorSubcoreMesh(core_axis_name="c", subcore_axis_name="s"),
    out_type=jax.ShapeDtypeStruct((B,), jnp.float32),
    compiler_params=pltpu.CompilerParams(needs_layout_passes=False,
                                         use_tc_tiling_on_sc=False),
    scratch_types=[
        pltpu.VMEM((BPW,), jnp.int32),
        pltpu.VMEM((BPW,), jnp.int32),
        pltpu.VMEM((BPW, D), jnp.float32),
        pltpu.VMEM((BPW, D), jnp.float32),
        pltpu.VMEM((L * L,), jnp.float32),
        pltpu.VMEM((L,), jnp.float32),
        pltpu.VMEM((L,), jnp.float32),
        pltpu.VMEM((BPW,), jnp.float32),
        pltpu.SemaphoreType.DMA,
    ],
)
def _sc_kernel(tgt_hbm, ctx_hbm, table_hbm, w_hbm, b_hbm, out_hbm,
               idx_t, idx_c, rows_t, rows_c, psum, wv, bv, out_v, sem):
    wid = lax.axis_index("s") * NC + lax.axis_index("c")
    base = wid * BPW

    pltpu.sync_copy(tgt_hbm.at[pl.ds(base, BPW)], idx_t)
    pltpu.sync_copy(ctx_hbm.at[pl.ds(base, BPW)], idx_c)
    pltpu.sync_copy(w_hbm, wv)
    pltpu.sync_copy(b_hbm, bv)

    copies = []
    for j in range(N_CHUNKS):
        sl = pl.ds(j * GATHER_CHUNK, GATHER_CHUNK)
        copies.append(pltpu.async_copy(table_hbm.at[idx_t.at[sl]],
                                       rows_t.at[sl], sem))
        copies.append(pltpu.async_copy(table_hbm.at[idx_c.at[sl]],
                                       rows_c.at[sl], sem))
    for c in copies:
        c.wait()

    iota = lax.iota(jnp.int32, L)
    w = wv[...]
    b = bv[...]
    one = jnp.ones((L,), jnp.float32)

    def group_body(g, carry):
        base_r = g * L
        for j in range(L):
            r = base_r + j
            pv = jnp.zeros((L,), jnp.float32)
            for k in range(D // L):
                vt = rows_t[r, pl.ds(k * L, L)]
                vc = rows_c[r, pl.ds(k * L, L)]
                pv = pv + vt * vc
            psum[pl.ds(j * L, L)] = pv
        acc = jnp.zeros((L,), jnp.float32)
        for l in range(L):
            acc = acc + plsc.load_gather(psum, [iota * L + l])
        x = acc * w + b
        y = one / (one + jnp.exp(-x))
        out_v[pl.ds(base_r, L)] = y
        return carry

    lax.fori_loop(0, N_GROUPS, group_body, 0)

    pltpu.sync_copy(out_v, out_hbm.at[pl.ds(base, BPW)])


def kernel(input_target, input_context, embedding_table, dense_w, dense_b):
    tgt = input_target.reshape(B).astype(jnp.int32)
    ctx = input_context.reshape(B).astype(jnp.int32)
    wv = jnp.broadcast_to(dense_w.reshape(1), (L,)).astype(jnp.float32)
    bv = jnp.broadcast_to(dense_b.reshape(1), (L,)).astype(jnp.float32)
    out = _sc_kernel(tgt, ctx, embedding_table, wv, bv)
    return out.reshape(B, 1)
